# baseline (speedup 1.0000x reference)
import jax
import jax.numpy as jnp
from jax import lax
from jax.experimental import pallas as pl
from jax.experimental.pallas import tpu as pltpu
from jax.experimental.pallas import tpu_sc as plsc

_NC, _NS, _L = 2, 16, 16
_NW = _NC * _NS
_ROWS, _COLS = 64, 32768
_SCROWS = 32
_TCROWS = _ROWS - _SCROWS
_CHUNKW = 2048
_CPR = _COLS // _CHUNKW
_NBUF = 4
_OPAD = 8
_UNROLL = 8


def _insert(m1, m2, m3, v):
    t1 = jnp.maximum(m1, v)
    v = jnp.minimum(m1, v)
    t2 = jnp.maximum(m2, v)
    v = jnp.minimum(m2, v)
    t3 = jnp.maximum(m3, v)
    return t1, t2, t3


def _pop_first(m1, m2, m3, g):
    eq = m1 == jnp.full((_L,), g)
    first = eq & (plsc.cumsum(eq.astype(jnp.int32)) == 1)
    return jnp.where(first, m2, m1), jnp.where(first, m3, m2), m3


def _sc_body(a_hbm, b_hbm, o_hbm, buf, bvec, ovec, sem0, sem1, sem2, sem3):
    wid = lax.axis_index("s") * _NC + lax.axis_index("c")
    neg = jnp.full((_L,), -jnp.inf, dtype=jnp.float32)
    sems = (sem0, sem1, sem2, sem3)

    def issue(k, slot):
        pltpu.async_copy(
            a_hbm.at[wid, pl.ds(k * _CHUNKW, _CHUNKW)],
            buf.at[pl.ds(slot * _CHUNKW, _CHUNKW)], sems[slot])

    for s in range(_NBUF):
        issue(s, s)

    pltpu.sync_copy(b_hbm, bvec.at[pl.ds(0, 3)])
    bv = bvec[...]
    b012 = (bv[0], bv[1], bv[2])

    def chunk_body(k, acc):
        slot = k % _NBUF
        for ss in range(_NBUF):
            @pl.when(slot == ss)
            def _slot(ss=ss):
                pltpu.make_async_copy(
                    a_hbm.at[wid, pl.ds(0, _CHUNKW)],
                    buf.at[pl.ds(0, _CHUNKW)], sems[ss]).wait()
        base = slot * _CHUNKW

        def vec_body(i, carry):
            a1, a2, a3, c1, c2, c3 = carry
            va = buf[pl.ds(base + i, _L)]
            vc = buf[pl.ds(base + i + _L, _L)]
            a1, a2, a3 = _insert(a1, a2, a3, va)
            c1, c2, c3 = _insert(c1, c2, c3, vc)
            return a1, a2, a3, c1, c2, c3

        acc = plsc.parallel_loop(
            0, _CHUNKW, 2 * _L, unroll=_UNROLL, carry=acc)(vec_body)

        @pl.when(k + _NBUF < _CPR)
        def _refill():
            for ss in range(_NBUF):
                @pl.when(slot == ss)
                def _re(ss=ss):
                    issue(k + _NBUF, ss)

        return acc

    acc = lax.fori_loop(0, _CPR, chunk_body, (neg, neg, neg, neg, neg, neg))

    a1, a2, a3, c1, c2, c3 = acc
    m1, m2, m3 = _insert(a1, a2, a3, c1)
    m1, m2, m3 = _insert(m1, m2, m3, c2)
    m1, m2, m3 = _insert(m1, m2, m3, c3)

    g1 = jnp.max(m1)
    m1, m2, m3 = _pop_first(m1, m2, m3, g1)
    g2 = jnp.max(m1)
    m1, m2, m3 = _pop_first(m1, m2, m3, g2)
    g3 = jnp.max(m1)

    lane = lax.iota(jnp.int32, _L)
    out = jnp.where(lane == 0, jnp.full((_L,), g1 * 2.0 + b012[0]),
          jnp.where(lane == 1, jnp.full((_L,), g2 * 2.0 + b012[1]),
                    jnp.full((_L,), g3 * 2.0 + b012[2])))
    ovec[...] = out
    pltpu.sync_copy(ovec.at[pl.ds(0, _OPAD)],
                    o_hbm.at[pl.ds(wid * _OPAD, _OPAD)])


def _tc_body(a_ref, b_ref, o_ref):
    x = a_ref[...]
    ids = lax.broadcasted_iota(jnp.int32, x.shape, 1)
    big = 2**30
    neg = float("-inf")

    m1 = jnp.max(x, axis=1, keepdims=True)
    i1 = jnp.min(jnp.where(x == m1, ids, big), axis=1, keepdims=True)
    x = jnp.where(ids == i1, neg, x)

    m2 = jnp.max(x, axis=1, keepdims=True)
    i2 = jnp.min(jnp.where(x == m2, ids, big), axis=1, keepdims=True)
    x = jnp.where(ids == i2, neg, x)

    m3 = jnp.max(x, axis=1, keepdims=True)

    c = lax.broadcasted_iota(jnp.int32, (x.shape[0], 128), 1)
    b0, b1, b2 = b_ref[0], b_ref[1], b_ref[2]
    vals = jnp.where(c == 0, m1 * 2.0 + b0,
           jnp.where(c == 1, m2 * 2.0 + b1,
           jnp.where(c == 2, m3 * 2.0 + b2, 0.0)))
    o_ref[...] = vals


@jax.jit
def kernel(a, b):
    sc_fn = pl.kernel(
        _sc_body,
        out_type=jax.ShapeDtypeStruct((_SCROWS * _OPAD,), jnp.float32),
        mesh=plsc.VectorSubcoreMesh(core_axis_name="c", subcore_axis_name="s"),
        compiler_params=pltpu.CompilerParams(needs_layout_passes=False),
        scratch_types=[
            pltpu.VMEM((_NBUF * _CHUNKW,), jnp.float32),
            pltpu.VMEM((_L,), jnp.float32),
            pltpu.VMEM((_L,), jnp.float32),
            pltpu.SemaphoreType.DMA,
            pltpu.SemaphoreType.DMA,
            pltpu.SemaphoreType.DMA,
            pltpu.SemaphoreType.DMA,
        ],
    )
    sc_out = sc_fn(a, b)

    tc_out = pl.pallas_call(
        _tc_body,
        grid=(1,),
        in_specs=[
            pl.BlockSpec((_TCROWS, _COLS), lambda i: (1, 0)),
            pl.BlockSpec(memory_space=pltpu.SMEM),
        ],
        out_specs=pl.BlockSpec((_TCROWS, 128), lambda i: (0, 0)),
        out_shape=jax.ShapeDtypeStruct((_TCROWS, 128), jnp.float32),
    )(a, b)

    return jnp.concatenate(
        [sc_out.reshape(_SCROWS, _OPAD)[:, :3], tc_out[:, :3]], axis=0)


# speedup vs baseline: 1.0295x; 1.0295x over previous
import jax
import jax.numpy as jnp
from jax import lax
from jax.experimental import pallas as pl
from jax.experimental.pallas import tpu as pltpu
from jax.experimental.pallas import tpu_sc as plsc

_NC, _NS, _L = 2, 16, 16
_NW = _NC * _NS
_ROWS, _COLS = 64, 32768
_SCROWS = 32
_TCROWS = _ROWS - _SCROWS
_CHUNKW = 4096
_CPR = _COLS // _CHUNKW
_NBUF = 4
_OPAD = 8
_UNROLL = 8


def _insert(m1, m2, m3, v):
    t1 = jnp.maximum(m1, v)
    v = jnp.minimum(m1, v)
    t2 = jnp.maximum(m2, v)
    v = jnp.minimum(m2, v)
    t3 = jnp.maximum(m3, v)
    return t1, t2, t3


def _pop_first(m1, m2, m3, g):
    eq = m1 == jnp.full((_L,), g)
    first = eq & (plsc.cumsum(eq.astype(jnp.int32)) == 1)
    return jnp.where(first, m2, m1), jnp.where(first, m3, m2), m3


def _sc_body(a_hbm, b_hbm, o_hbm, buf, bvec, ovec, sem0, sem1, sem2, sem3):
    wid = lax.axis_index("s") * _NC + lax.axis_index("c")
    neg = jnp.full((_L,), -jnp.inf, dtype=jnp.float32)
    sems = (sem0, sem1, sem2, sem3)

    def issue(k, slot):
        pltpu.async_copy(
            a_hbm.at[wid, pl.ds(k * _CHUNKW, _CHUNKW)],
            buf.at[pl.ds(slot * _CHUNKW, _CHUNKW)], sems[slot])

    for s in range(_NBUF):
        issue(s, s)

    pltpu.sync_copy(b_hbm, bvec.at[pl.ds(0, 3)])
    bv = bvec[...]
    b012 = (bv[0], bv[1], bv[2])

    def chunk_body(k, acc):
        slot = k % _NBUF
        for ss in range(_NBUF):
            @pl.when(slot == ss)
            def _slot(ss=ss):
                pltpu.make_async_copy(
                    a_hbm.at[wid, pl.ds(0, _CHUNKW)],
                    buf.at[pl.ds(0, _CHUNKW)], sems[ss]).wait()
        base = slot * _CHUNKW

        def vec_body(i, carry):
            a1, a2, a3, c1, c2, c3 = carry
            va = buf[pl.ds(base + i, _L)]
            vc = buf[pl.ds(base + i + _L, _L)]
            a1, a2, a3 = _insert(a1, a2, a3, va)
            c1, c2, c3 = _insert(c1, c2, c3, vc)
            return a1, a2, a3, c1, c2, c3

        acc = plsc.parallel_loop(
            0, _CHUNKW, 2 * _L, unroll=_UNROLL, carry=acc)(vec_body)

        @pl.when(k + _NBUF < _CPR)
        def _refill():
            for ss in range(_NBUF):
                @pl.when(slot == ss)
                def _re(ss=ss):
                    issue(k + _NBUF, ss)

        return acc

    acc = lax.fori_loop(0, _CPR, chunk_body, (neg, neg, neg, neg, neg, neg))

    a1, a2, a3, c1, c2, c3 = acc
    m1, m2, m3 = _insert(a1, a2, a3, c1)
    m1, m2, m3 = _insert(m1, m2, m3, c2)
    m1, m2, m3 = _insert(m1, m2, m3, c3)

    g1 = jnp.max(m1)
    m1, m2, m3 = _pop_first(m1, m2, m3, g1)
    g2 = jnp.max(m1)
    m1, m2, m3 = _pop_first(m1, m2, m3, g2)
    g3 = jnp.max(m1)

    lane = lax.iota(jnp.int32, _L)
    out = jnp.where(lane == 0, jnp.full((_L,), g1 * 2.0 + b012[0]),
          jnp.where(lane == 1, jnp.full((_L,), g2 * 2.0 + b012[1]),
                    jnp.full((_L,), g3 * 2.0 + b012[2])))
    ovec[...] = out
    pltpu.sync_copy(ovec.at[pl.ds(0, _OPAD)],
                    o_hbm.at[pl.ds(wid * _OPAD, _OPAD)])


def _tc_body(a_ref, b_ref, o_ref):
    x = a_ref[...]
    ids = lax.broadcasted_iota(jnp.int32, x.shape, 1)
    big = 2**30
    neg = float("-inf")

    m1 = jnp.max(x, axis=1, keepdims=True)
    i1 = jnp.min(jnp.where(x == m1, ids, big), axis=1, keepdims=True)
    x = jnp.where(ids == i1, neg, x)

    m2 = jnp.max(x, axis=1, keepdims=True)
    i2 = jnp.min(jnp.where(x == m2, ids, big), axis=1, keepdims=True)
    x = jnp.where(ids == i2, neg, x)

    m3 = jnp.max(x, axis=1, keepdims=True)

    c = lax.broadcasted_iota(jnp.int32, (x.shape[0], 128), 1)
    b0, b1, b2 = b_ref[0], b_ref[1], b_ref[2]
    vals = jnp.where(c == 0, m1 * 2.0 + b0,
           jnp.where(c == 1, m2 * 2.0 + b1,
           jnp.where(c == 2, m3 * 2.0 + b2, 0.0)))
    o_ref[...] = vals


@jax.jit
def kernel(a, b):
    sc_fn = pl.kernel(
        _sc_body,
        out_type=jax.ShapeDtypeStruct((_SCROWS * _OPAD,), jnp.float32),
        mesh=plsc.VectorSubcoreMesh(core_axis_name="c", subcore_axis_name="s"),
        compiler_params=pltpu.CompilerParams(needs_layout_passes=False),
        scratch_types=[
            pltpu.VMEM((_NBUF * _CHUNKW,), jnp.float32),
            pltpu.VMEM((_L,), jnp.float32),
            pltpu.VMEM((_L,), jnp.float32),
            pltpu.SemaphoreType.DMA,
            pltpu.SemaphoreType.DMA,
            pltpu.SemaphoreType.DMA,
            pltpu.SemaphoreType.DMA,
        ],
    )
    tc_out = pl.pallas_call(
        _tc_body,
        grid=(1,),
        in_specs=[
            pl.BlockSpec((_TCROWS, _COLS), lambda i: (1, 0)),
            pl.BlockSpec(memory_space=pltpu.SMEM),
        ],
        out_specs=pl.BlockSpec((_TCROWS, 128), lambda i: (0, 0)),
        out_shape=jax.ShapeDtypeStruct((_TCROWS, 128), jnp.float32),
    )(a, b)

    sc_out = sc_fn(a, b)

    return jnp.concatenate(
        [sc_out.reshape(_SCROWS, _OPAD)[:, :3], tc_out[:, :3]], axis=0)


# speedup vs baseline: 1.0415x; 1.0117x over previous
import jax
import jax.numpy as jnp
from jax import lax
from jax.experimental import pallas as pl
from jax.experimental.pallas import tpu as pltpu
from jax.experimental.pallas import tpu_sc as plsc

_NC, _NS, _L = 2, 16, 16
_NW = _NC * _NS
_ROWS, _COLS = 64, 32768
_SCROWS = 32
_TCROWS = _ROWS - _SCROWS
_CHUNKW = 4096
_CPR = _COLS // _CHUNKW
_NBUF = 4
_OPAD = 8
_UNROLL = 8


def _insert(m1, m2, m3, v):
    t1 = jnp.maximum(m1, v)
    v = jnp.minimum(m1, v)
    t2 = jnp.maximum(m2, v)
    v = jnp.minimum(m2, v)
    t3 = jnp.maximum(m3, v)
    return t1, t2, t3


def _pop_first(m1, m2, m3, g):
    eq = m1 == jnp.full((_L,), g)
    first = eq & (plsc.cumsum(eq.astype(jnp.int32)) == 1)
    return jnp.where(first, m2, m1), jnp.where(first, m3, m2), m3


def _sc_body(a_hbm, b_hbm, o_hbm, buf, bvec, ovec, sem0, sem1, sem2, sem3):
    wid = lax.axis_index("s") * _NC + lax.axis_index("c")
    neg = jnp.full((_L,), -jnp.inf, dtype=jnp.float32)
    sems = (sem0, sem1, sem2, sem3)

    def issue(k, slot):
        pltpu.async_copy(
            a_hbm.at[wid, pl.ds(k * _CHUNKW, _CHUNKW)],
            buf.at[pl.ds(slot * _CHUNKW, _CHUNKW)], sems[slot])

    for s in range(_NBUF):
        issue(s, s)

    pltpu.sync_copy(b_hbm, bvec.at[pl.ds(0, 3)])
    bv = bvec[...]
    b012 = (bv[0], bv[1], bv[2])

    def chunk_body(k, acc):
        slot = k % _NBUF
        for ss in range(_NBUF):
            @pl.when(slot == ss)
            def _slot(ss=ss):
                pltpu.make_async_copy(
                    a_hbm.at[wid, pl.ds(0, _CHUNKW)],
                    buf.at[pl.ds(0, _CHUNKW)], sems[ss]).wait()
        base = slot * _CHUNKW

        def vec_body(i, carry):
            a1, a2, a3, c1, c2, c3 = carry
            va = buf[pl.ds(base + i, _L)]
            vc = buf[pl.ds(base + i + _L, _L)]
            a1, a2, a3 = _insert(a1, a2, a3, va)
            c1, c2, c3 = _insert(c1, c2, c3, vc)
            return a1, a2, a3, c1, c2, c3

        acc = plsc.parallel_loop(
            0, _CHUNKW, 2 * _L, unroll=_UNROLL, carry=acc)(vec_body)

        @pl.when(k + _NBUF < _CPR)
        def _refill():
            for ss in range(_NBUF):
                @pl.when(slot == ss)
                def _re(ss=ss):
                    issue(k + _NBUF, ss)

        return acc

    acc = lax.fori_loop(0, _CPR, chunk_body, (neg, neg, neg, neg, neg, neg))

    a1, a2, a3, c1, c2, c3 = acc
    m1, m2, m3 = _insert(a1, a2, a3, c1)
    m1, m2, m3 = _insert(m1, m2, m3, c2)
    m1, m2, m3 = _insert(m1, m2, m3, c3)

    g1 = jnp.max(m1)
    m1, m2, m3 = _pop_first(m1, m2, m3, g1)
    g2 = jnp.max(m1)
    m1, m2, m3 = _pop_first(m1, m2, m3, g2)
    g3 = jnp.max(m1)

    lane = lax.iota(jnp.int32, _L)
    out = jnp.where(lane == 0, jnp.full((_L,), g1 * 2.0 + b012[0]),
          jnp.where(lane == 1, jnp.full((_L,), g2 * 2.0 + b012[1]),
                    jnp.full((_L,), g3 * 2.0 + b012[2])))
    ovec[...] = out
    pltpu.sync_copy(ovec.at[pl.ds(0, _OPAD)],
                    o_hbm.at[pl.ds(wid * _OPAD, _OPAD)])


_TCBLK = 4096
_TCGRID = _COLS // _TCBLK


def _tc_pop(m1, m2, m3, ids, big):
    g = jnp.max(m1, axis=1, keepdims=True)
    j = jnp.min(jnp.where(m1 == g, ids, big), axis=1, keepdims=True)
    sel = ids == j
    return g, jnp.where(sel, m2, m1), jnp.where(sel, m3, m2)


def _tc_body(a_ref, b_ref, o_ref, m1_ref, m2_ref, m3_ref):
    step = pl.program_id(0)
    neg = float("-inf")
    big = 2**30

    @pl.when(step == 0)
    def _init():
        full = jnp.full((_TCROWS, 128), neg, dtype=jnp.float32)
        m1_ref[...] = full
        m2_ref[...] = full
        m3_ref[...] = full

    x = a_ref[...]
    m1, m2, m3 = m1_ref[...], m2_ref[...], m3_ref[...]
    for j in range(_TCBLK // 128):
        v = x[:, j * 128:(j + 1) * 128]
        t1 = jnp.maximum(m1, v)
        v = jnp.minimum(m1, v)
        t2 = jnp.maximum(m2, v)
        v = jnp.minimum(m2, v)
        m3 = jnp.maximum(m3, v)
        m1, m2 = t1, t2
    m1_ref[...] = m1
    m2_ref[...] = m2
    m3_ref[...] = m3

    @pl.when(step == _TCGRID - 1)
    def _fin():
        ids = lax.broadcasted_iota(jnp.int32, (_TCROWS, 128), 1)
        g1, p1, p2 = _tc_pop(m1, m2, m3, ids, big)
        g2, q1, _ = _tc_pop(p1, p2, m3, ids, big)
        g3 = jnp.max(q1, axis=1, keepdims=True)
        b0, b1, b2 = b_ref[0], b_ref[1], b_ref[2]
        vals = jnp.where(ids == 0, g1 * 2.0 + b0,
               jnp.where(ids == 1, g2 * 2.0 + b1,
               jnp.where(ids == 2, g3 * 2.0 + b2, 0.0)))
        o_ref[...] = vals


@jax.jit
def kernel(a, b):
    sc_fn = pl.kernel(
        _sc_body,
        out_type=jax.ShapeDtypeStruct((_SCROWS * _OPAD,), jnp.float32),
        mesh=plsc.VectorSubcoreMesh(core_axis_name="c", subcore_axis_name="s"),
        compiler_params=pltpu.CompilerParams(needs_layout_passes=False),
        scratch_types=[
            pltpu.VMEM((_NBUF * _CHUNKW,), jnp.float32),
            pltpu.VMEM((_L,), jnp.float32),
            pltpu.VMEM((_L,), jnp.float32),
            pltpu.SemaphoreType.DMA,
            pltpu.SemaphoreType.DMA,
            pltpu.SemaphoreType.DMA,
            pltpu.SemaphoreType.DMA,
        ],
    )
    tc_out = pl.pallas_call(
        _tc_body,
        grid=(_TCGRID,),
        in_specs=[
            pl.BlockSpec((_TCROWS, _TCBLK), lambda i: (1, i)),
            pl.BlockSpec(memory_space=pltpu.SMEM),
        ],
        out_specs=pl.BlockSpec((_TCROWS, 128), lambda i: (0, 0)),
        out_shape=jax.ShapeDtypeStruct((_TCROWS, 128), jnp.float32),
        scratch_shapes=[
            pltpu.VMEM((_TCROWS, 128), jnp.float32),
            pltpu.VMEM((_TCROWS, 128), jnp.float32),
            pltpu.VMEM((_TCROWS, 128), jnp.float32),
        ],
    )(a, b)

    sc_out = sc_fn(a, b)

    return jnp.concatenate(
        [sc_out.reshape(_SCROWS, _OPAD)[:, :3], tc_out[:, :3]], axis=0)
